# baseline (speedup 1.0000x reference)
import functools
import jax
import jax.numpy as jnp
from jax import lax
from jax.experimental import pallas as pl
from jax.experimental.pallas import tpu as pltpu
from jax.experimental.pallas import tpu_sc as plsc

VOCAB = 32768
MASK = 32767
N_ROWS = 2048
LANES = 16
NW = 32
ROWS_PER_W = N_ROWS // NW
NSLOT = 2 * ROWS_PER_W
BV = 4096


def _zero_block(out_ref):
    out_ref[...] = jnp.zeros_like(out_ref)


def _sc_scatter(ids_hbm, ls_hbm, out_ref, ids_v, ls_v, val_v, idx_v, sem):
    wid = lax.axis_index("s") * 2 + lax.axis_index("c")
    row0 = wid * ROWS_PER_W

    pltpu.sync_copy(ids_hbm.at[pl.ds(row0, ROWS_PER_W)], ids_v)
    pltpu.sync_copy(ls_hbm.at[pl.ds(row0, ROWS_PER_W)], ls_v)

    one = jnp.full((LANES,), 1.0, jnp.float32)
    for c in range(ROWS_PER_W // LANES):
        ids16 = ids_v[pl.ds(c * LANES, LANES)]
        ls16 = ls_v[pl.ds(c * LANES, LANES)]
        x = jnp.minimum(jnp.maximum(ls16, -10.0), 10.0)
        alpha = 1.0 / (1.0 + jnp.exp(-x))
        is_mask = ids16 == jnp.full((LANES,), MASK, jnp.int32)
        row = (jnp.full((LANES,), row0 + c * LANES, jnp.int32)
               + lax.iota(jnp.int32, LANES)) * VOCAB
        val_v[pl.ds(c * 2 * LANES, LANES)] = jnp.where(is_mask, one, alpha)
        val_v[pl.ds((c * 2 + 1) * LANES, LANES)] = jnp.where(is_mask, one, 1.0 - alpha)
        idx_v[pl.ds(c * 2 * LANES, LANES)] = row + ids16
        idx_v[pl.ds((c * 2 + 1) * LANES, LANES)] = row + jnp.full(
            (LANES,), MASK, jnp.int32)

    pltpu.async_copy(val_v, out_ref.at[idx_v], sem).wait()


def kernel(log_snr, input_ids):
    B, L = log_snr.shape
    ids_flat = input_ids.astype(jnp.int32).reshape(-1)
    ls_flat = log_snr.reshape(-1)

    zeros = pl.pallas_call(
        _zero_block,
        grid=(B, VOCAB // BV),
        out_specs=pl.BlockSpec((L, BV), lambda i, j: (i, j)),
        out_shape=jax.ShapeDtypeStruct((N_ROWS, VOCAB), jnp.float32),
        compiler_params=pltpu.CompilerParams(
            dimension_semantics=("arbitrary", "arbitrary"),
        ),
    )()

    mesh = plsc.VectorSubcoreMesh(
        core_axis_name="c", subcore_axis_name="s", num_cores=2, num_subcores=16)
    sc_k = functools.partial(
        pl.kernel,
        mesh=mesh,
        out_type=(),
        scratch_types=[
            pltpu.VMEM((ROWS_PER_W,), jnp.int32),
            pltpu.VMEM((ROWS_PER_W,), jnp.float32),
            pltpu.VMEM((NSLOT,), jnp.float32),
            pltpu.VMEM((NSLOT,), jnp.int32),
            pltpu.SemaphoreType.DMA,
        ],
    )(_sc_scatter)

    out_ref = jax.new_ref(zeros.reshape(N_ROWS * VOCAB))
    sc_k(ids_flat, ls_flat, out_ref)
    return jax.freeze(out_ref).reshape(B, L, VOCAB)


# speedup vs baseline: 5.0655x; 5.0655x over previous
import functools
import jax
import jax.numpy as jnp
from jax import lax
from jax.experimental import pallas as pl
from jax.experimental.pallas import tpu as pltpu
from jax.experimental.pallas import tpu_sc as plsc

VOCAB = 32768
MASK = 32767
N_ROWS = 2048
LANES = 16
NW = 32
ROWS_PER_W = N_ROWS // NW
ZC = 8192
NQ = VOCAB // ZC


def _sc_body(ids_hbm, ls_hbm, out_hbm, ids_v, ls_v, z_v, sem):
    wid = lax.axis_index("s") * 2 + lax.axis_index("c")
    row0 = wid * ROWS_PER_W

    pltpu.sync_copy(ids_hbm.at[pl.ds(row0, ROWS_PER_W)], ids_v)
    pltpu.sync_copy(ls_hbm.at[pl.ds(row0, ROWS_PER_W)], ls_v)

    iota = lax.iota(jnp.int32, LANES)

    id_s = []
    a_s = []
    for c in range(ROWS_PER_W // LANES):
        ids16 = ids_v[pl.ds(c * LANES, LANES)]
        ls16 = ls_v[pl.ds(c * LANES, LANES)]
        x = jnp.minimum(jnp.maximum(ls16, -10.0), 10.0)
        alpha16 = 1.0 / (1.0 + jnp.exp(-x))
        for t in range(LANES):
            lane_mask = iota == t
            id_s.append(jnp.max(jnp.where(lane_mask, ids16, -1)))
            a_s.append(jnp.max(jnp.where(lane_mask, alpha16, -1.0)))

    zero16 = jnp.zeros((LANES,), jnp.float32)

    def _zrow(s, carry):
        for rloc in range(8):
            z_v[rloc, pl.ds(s * LANES, LANES)] = zero16
        return carry

    lax.fori_loop(0, ZC // LANES, _zrow, None)

    for k in range(ROWS_PER_W // 8):
        for q in range(NQ):
            for rloc in range(8):
                j = k * 8 + rloc
                id_j = id_s[j]
                a_j = a_s[j]
                w_j = jnp.where(id_j == MASK, 1.0, a_j)
                wm_j = 1.0 - a_j
                if q == NQ - 1:
                    z_v[rloc, pl.ds(ZC - LANES, LANES)] = jnp.where(
                        iota == 15, jnp.full((LANES,), wm_j, jnp.float32),
                        0.0)
                in_q = (id_j // ZC) == q
                idq = id_j - q * ZC
                s16 = (idq // LANES) * LANES
                ln = idq - s16
                if q == NQ - 1:
                    wtail_j = jnp.where(idq >= ZC - LANES, wm_j, 0.0)
                else:
                    wtail_j = jnp.float32(0.0)

                @pl.when(in_q)
                def _():
                    span = jnp.where(
                        iota == ln, jnp.full((LANES,), w_j, jnp.float32),
                        jnp.where(iota == 15,
                                  jnp.full((LANES,), wtail_j, jnp.float32),
                                  0.0))
                    z_v[rloc, pl.ds(s16, LANES)] = span

            pltpu.sync_copy(
                z_v, out_hbm.at[pl.ds(row0 + k * 8, 8), pl.ds(q * ZC, ZC)])

            for rloc in range(8):
                j = k * 8 + rloc
                id_j = id_s[j]
                if q == NQ - 1:
                    z_v[rloc, pl.ds(ZC - LANES, LANES)] = zero16
                in_q = (id_j // ZC) == q
                idq = id_j - q * ZC
                s16 = (idq // LANES) * LANES

                @pl.when(in_q)
                def _():
                    z_v[rloc, pl.ds(s16, LANES)] = zero16


@jax.jit
def _run(ids_flat, ls_flat):
    mesh = plsc.VectorSubcoreMesh(
        core_axis_name="c", subcore_axis_name="s", num_cores=2, num_subcores=16)
    k = functools.partial(
        pl.kernel,
        mesh=mesh,
        out_type=jax.ShapeDtypeStruct((N_ROWS, VOCAB), jnp.float32),
        scratch_types=[
            pltpu.VMEM((ROWS_PER_W,), jnp.int32),
            pltpu.VMEM((ROWS_PER_W,), jnp.float32),
            pltpu.VMEM((8, ZC), jnp.float32),
            pltpu.SemaphoreType.DMA,
        ],
        compiler_params=pltpu.CompilerParams(use_tc_tiling_on_sc=True, needs_layout_passes=False),
    )(_sc_body)
    return k(ids_flat, ls_flat)


def kernel(log_snr, input_ids):
    B, L = log_snr.shape
    ids_flat = input_ids.astype(jnp.int32).reshape(-1)
    ls_flat = log_snr.reshape(-1)
    out = _run(ids_flat, ls_flat)
    return out.reshape(B, L, VOCAB)
